# baseline (speedup 1.0000x reference)
import functools

import jax
import jax.numpy as jnp
from jax import lax
from jax.experimental import pallas as pl
from jax.experimental.pallas import tpu as pltpu
from jax.experimental.pallas import tpu_sc as plsc

_B, _S, _D = 16, 2048, 128
_ROWS_OUT = _B * (_S // 2)
_NC, _NS = 2, 16
_NW = _NC * _NS
_RPW = _ROWS_OUT // _NW


def _sampler_body(trip_hbm, out_hbm, sem):
    wid = lax.axis_index("s") * _NC + lax.axis_index("c")
    base = wid * _RPW
    pltpu.async_copy(
        trip_hbm.at[pl.ds(base, _RPW), pl.ds(0, _D)],
        out_hbm.at[pl.ds(base, _RPW)],
        sem,
    ).wait()


@jax.jit
def _sampler(trip2d):
    mesh = plsc.VectorSubcoreMesh(core_axis_name="c", subcore_axis_name="s")
    k = pl.kernel(
        _sampler_body,
        out_type=jax.ShapeDtypeStruct((_ROWS_OUT, _D), jnp.float32),
        mesh=mesh,
        scratch_types=[
            pltpu.SemaphoreType.DMA,
        ],
    )
    return k(trip2d)


def kernel(trip):
    trip_wide = trip.reshape(_ROWS_OUT, 2 * _D)
    out2d = _sampler(trip_wide)
    return out2d.reshape(_B, _S // 2, _D)


# speedup vs baseline: 6.9070x; 6.9070x over previous
import functools

import jax
import jax.numpy as jnp
from jax import lax
from jax.experimental import pallas as pl
from jax.experimental.pallas import tpu as pltpu
from jax.experimental.pallas import tpu_sc as plsc

_B, _S, _D = 16, 2048, 128
_ROWS_OUT = _B * (_S // 2)
_NC, _NS = 2, 16
_NW = _NC * _NS
_RPW = _ROWS_OUT // _NW


_CHUNK = 128
_NCHUNK = _RPW // _CHUNK


def _sampler_body(trip_hbm, out_hbm, rows_v, gsem, wsem):
    wid = lax.axis_index("s") * _NC + lax.axis_index("c")
    base = wid * _RPW

    gathers = []
    for j in range(_NCHUNK):
        gathers.append(
            pltpu.async_copy(
                trip_hbm.at[pl.ds(base + j * _CHUNK, _CHUNK), pl.ds(0, _D)],
                rows_v.at[pl.ds(j * _CHUNK, _CHUNK)],
                gsem,
            )
        )
    writes = []
    for j in range(_NCHUNK):
        gathers[j].wait()
        writes.append(
            pltpu.async_copy(
                rows_v.at[pl.ds(j * _CHUNK, _CHUNK)],
                out_hbm.at[pl.ds(base + j * _CHUNK, _CHUNK)],
                wsem,
            )
        )
    for w in writes:
        w.wait()


@jax.jit
def _sampler(trip2d):
    mesh = plsc.VectorSubcoreMesh(core_axis_name="c", subcore_axis_name="s")
    k = pl.kernel(
        _sampler_body,
        out_type=jax.ShapeDtypeStruct((_ROWS_OUT, _D), jnp.float32),
        mesh=mesh,
        scratch_types=[
            pltpu.VMEM((_RPW, _D), jnp.float32),
            pltpu.SemaphoreType.DMA,
            pltpu.SemaphoreType.DMA,
        ],
    )
    return k(trip2d)


def kernel(trip):
    trip_wide = trip.reshape(_ROWS_OUT, 2 * _D)
    out2d = _sampler(trip_wide)
    return out2d.reshape(_B, _S // 2, _D)


# speedup vs baseline: 11.4260x; 1.6543x over previous
import jax
import jax.numpy as jnp
from jax import lax
from jax.experimental import pallas as pl
from jax.experimental.pallas import tpu as pltpu
from jax.experimental.pallas import tpu_sc as plsc

_B, _S, _D = 16, 2048, 128
_ROWS_OUT = _B * (_S // 2)
_NC, _NS, _L = 2, 16, 16
_NW = _NC * _NS
_RPW = _ROWS_OUT // _NW
_CHUNK = 64
_NCHUNK = _RPW // _CHUNK


def _sampler_body(trip_hbm, out_hbm, idx_v, rows_v, gsem, wsem):
    wid = lax.axis_index("s") * _NC + lax.axis_index("c")
    base = wid * _RPW

    iota = lax.iota(jnp.int32, _L)
    for j in range(_NCHUNK):
        for i in range(_CHUNK // _L):
            start = base + j * _CHUNK + i * _L
            idx_v[j, pl.ds(i * _L, _L)] = 2 * start + 2 * iota

    gathers = []
    for j in range(_NCHUNK):
        gathers.append(
            pltpu.async_copy(
                trip_hbm.at[idx_v.at[j]],
                rows_v.at[pl.ds(j * _CHUNK, _CHUNK)],
                gsem,
            )
        )
    writes = []
    for j in range(_NCHUNK):
        gathers[j].wait()
        writes.append(
            pltpu.async_copy(
                rows_v.at[pl.ds(j * _CHUNK, _CHUNK)],
                out_hbm.at[pl.ds(base + j * _CHUNK, _CHUNK)],
                wsem,
            )
        )
    for w in writes:
        w.wait()


@jax.jit
def _sampler(trip2d):
    mesh = plsc.VectorSubcoreMesh(core_axis_name="c", subcore_axis_name="s")
    k = pl.kernel(
        _sampler_body,
        out_type=jax.ShapeDtypeStruct((_ROWS_OUT, _D), jnp.float32),
        mesh=mesh,
        scratch_types=[
            pltpu.VMEM((_NCHUNK, _CHUNK), jnp.int32),
            pltpu.VMEM((_RPW, _D), jnp.float32),
            pltpu.SemaphoreType.DMA,
            pltpu.SemaphoreType.DMA,
        ],
    )
    return k(trip2d)


def kernel(trip):
    trip2d = trip.reshape(_B * _S, _D)
    out2d = _sampler(trip2d)
    return out2d.reshape(_B, _S // 2, _D)
